# baseline (speedup 1.0000x reference)
import jax
import jax.numpy as jnp
from jax import lax
from jax.experimental import pallas as pl
from jax.experimental.pallas import tpu as pltpu
from jax.experimental.pallas import tpu_sc as plsc

NUM_FACTORS = 32
BATCH = 16384
NC = 2
NS = 16
L = 16
NW = NC * NS
B_PER_W = BATCH // NW
CHUNK = 128
NCHUNK = B_PER_W // CHUNK
NGROUP = B_PER_W // L


def _gmf_body(uidx_hbm, iidx_hbm, utab_hbm, itab_hbm, w_hbm, b_hbm, out_hbm,
              uidx_v, iidx_v, urows_v, irows_v, w_v, b_v, t_v, out_v, sem):
    wid = lax.axis_index("s") * NC + lax.axis_index("c")
    base = wid * B_PER_W

    for j in range(NCHUNK):
        pltpu.sync_copy(uidx_hbm.at[pl.ds(base + j * CHUNK, CHUNK)], uidx_v.at[j])
        pltpu.sync_copy(iidx_hbm.at[pl.ds(base + j * CHUNK, CHUNK)], iidx_v.at[j])
    pltpu.sync_copy(w_hbm, w_v)
    pltpu.sync_copy(b_hbm, b_v)

    copies = []
    for j in range(NCHUNK):
        dst = pl.ds(j * CHUNK, CHUNK)
        copies.append(pltpu.async_copy(utab_hbm.at[uidx_v.at[j]],
                                       urows_v.at[dst], sem))
        copies.append(pltpu.async_copy(itab_hbm.at[iidx_v.at[j]],
                                       irows_v.at[dst], sem))
    for c in copies:
        c.wait()

    w0 = w_v[pl.ds(0, L)]
    w1 = w_v[pl.ds(L, L)]
    bias = b_v[...]
    lane = lax.iota(jnp.int32, L)

    def group(g, carry):
        rbase = g * L
        for b in range(L):
            r = rbase + b
            u0 = urows_v[r, pl.ds(0, L)]
            u1 = urows_v[r, pl.ds(L, L)]
            i0 = irows_v[r, pl.ds(0, L)]
            i1 = irows_v[r, pl.ds(L, L)]
            p = u0 * i0 * w0 + u1 * i1 * w1
            plsc.store_scatter(t_v, [lane * L + b], p)
        acc = t_v[pl.ds(0, L)]
        for k in range(1, L):
            acc = acc + t_v[pl.ds(k * L, L)]
        x = acc + bias
        out_v[pl.ds(rbase, L)] = 1.0 / (1.0 + jnp.exp(-x))
        return carry

    lax.fori_loop(0, NGROUP, group, 0)

    pltpu.sync_copy(out_v, out_hbm.at[pl.ds(base, B_PER_W)])


_gmf = pl.kernel(
    _gmf_body,
    out_type=jax.ShapeDtypeStruct((BATCH,), jnp.float32),
    mesh=plsc.VectorSubcoreMesh(core_axis_name="c", subcore_axis_name="s",
                                num_cores=NC, num_subcores=NS),
    compiler_params=pltpu.CompilerParams(needs_layout_passes=False,
                                         use_tc_tiling_on_sc=False),
    scratch_types=[
        pltpu.VMEM((NCHUNK, CHUNK), jnp.int32),
        pltpu.VMEM((NCHUNK, CHUNK), jnp.int32),
        pltpu.VMEM((B_PER_W, NUM_FACTORS), jnp.float32),
        pltpu.VMEM((B_PER_W, NUM_FACTORS), jnp.float32),
        pltpu.VMEM((NUM_FACTORS,), jnp.float32),
        pltpu.VMEM((L,), jnp.float32),
        pltpu.VMEM((L * L,), jnp.float32),
        pltpu.VMEM((B_PER_W,), jnp.float32),
        pltpu.SemaphoreType.DMA,
    ],
)


def kernel(user_indices, item_indices, user_table, item_table, affine_w, affine_b):
    uidx = user_indices.astype(jnp.int32)
    iidx = item_indices.astype(jnp.int32)
    w_flat = affine_w.reshape(NUM_FACTORS)
    b_b = jnp.broadcast_to(affine_b.reshape(1), (L,))
    return _gmf(uidx, iidx, user_table, item_table, w_flat, b_b)


# speedup vs baseline: 1.6811x; 1.6811x over previous
import jax
import jax.numpy as jnp
from jax import lax
from jax.experimental import pallas as pl
from jax.experimental.pallas import tpu as pltpu
from jax.experimental.pallas import tpu_sc as plsc

NUM_FACTORS = 32
BATCH = 16384
NC = 2
NS = 16
L = 16
NW = NC * NS
B_PER_W = BATCH // NW
CHUNK = 128
NCHUNK = B_PER_W // CHUNK
NGROUP = B_PER_W // L
PACK = 128 // NUM_FACTORS
N4 = 249856
NMAIN = PACK * N4
NTAIL = (1000000 - NMAIN) // PACK
PC = 7808
PGRID = N4 // PC
HCHUNK = 2
HB = HCHUNK * CHUNK
HGROUP = HB // L
STAGE = HB + NTAIL


def _pack_body(x0_ref, x1_ref, x2_ref, x3_ref, o_ref):
    o_ref[...] = jnp.concatenate(
        [x0_ref[...].T, x1_ref[...].T, x2_ref[...].T, x3_ref[...].T], axis=1)


_pack = pl.pallas_call(
    _pack_body,
    grid=(PGRID,),
    in_specs=[
        pl.BlockSpec((NUM_FACTORS, PC), lambda i, s=s: (0, s * (N4 // PC) + i))
        for s in range(PACK)
    ],
    out_specs=pl.BlockSpec((PC, 128), lambda i: (i, 0)),
    out_shape=jax.ShapeDtypeStruct((N4, 128), jnp.float32),
)


def _gmf_body(uidx_hbm, iidx_hbm, utab_hbm, itab_hbm, utail_hbm, itail_hbm,
              w_hbm, b_hbm, out_hbm,
              uidx_v, iidx_v, qu_v, qi_v, ulines_v, ilines_v, w_v, b_v,
              out_v, sem):
    wid = lax.axis_index("s") * NC + lax.axis_index("c")
    base = wid * B_PER_W

    for j in range(NCHUNK):
        pltpu.sync_copy(uidx_hbm.at[pl.ds(base + j * CHUNK, CHUNK)], uidx_v.at[j])
        pltpu.sync_copy(iidx_hbm.at[pl.ds(base + j * CHUNK, CHUNK)], iidx_v.at[j])
    pltpu.sync_copy(w_hbm, w_v)
    pltpu.sync_copy(b_hbm, b_v)
    pltpu.sync_copy(utail_hbm, ulines_v.at[pl.ds(HB, NTAIL)])
    pltpu.sync_copy(itail_hbm, ilines_v.at[pl.ds(HB, NTAIL)])

    def quarter(v):
        return ((v >= N4).astype(jnp.int32) + (v >= 2 * N4).astype(jnp.int32)
                + (v >= 3 * N4).astype(jnp.int32))

    def qprep(g, carry):
        sl = pl.ds(g * L, L)
        for (src, dst) in ((uidx_v, qu_v), (iidx_v, qi_v)):
            v = src[g // (CHUNK // L), pl.ds((g % (CHUNK // L)) * L, L)]
            dst[sl] = jnp.minimum(v - quarter(v) * N4, N4 - 1)
        return carry

    lax.fori_loop(0, NGROUP, qprep, 0)

    ws = [w_v[pl.ds(f * L, L)] for f in range(NUM_FACTORS)]
    bias = b_v[...]
    lane = lax.iota(jnp.int32, L)

    for h in range(2):
        copies = []
        for jj in range(HCHUNK):
            j = h * HCHUNK + jj
            dst = pl.ds(jj * CHUNK, CHUNK)
            copies.append(pltpu.async_copy(
                utab_hbm.at[qu_v.at[pl.ds(j * CHUNK, CHUNK)]],
                ulines_v.at[dst], sem))
            copies.append(pltpu.async_copy(
                itab_hbm.at[qi_v.at[pl.ds(j * CHUNK, CHUNK)]],
                ilines_v.at[dst], sem))
        for c in copies:
            c.wait()

        def group(g, carry):
            gg = h * HGROUP + g
            ci, cs = gg // (CHUNK // L), pl.ds((gg % (CHUNK // L)) * L, L)
            local = g * L + lane
            acc = jnp.zeros((L,), jnp.float32)
            for (idxv, lines) in ((uidx_v, ulines_v), (iidx_v, ilines_v)):
                tid = idxv[ci, cs]
                tail = tid >= PACK * N4
                toff = tid - PACK * N4
                row = jnp.where(tail, HB + lax.shift_right_logical(toff, 2),
                                local)
                sub = jnp.where(tail, (toff & (PACK - 1)) * NUM_FACTORS,
                                quarter(tid) * NUM_FACTORS)
                if lines is ulines_v:
                    urow, su = row, sub
                else:
                    irow, si = row, sub
            for f in range(NUM_FACTORS):
                u = plsc.load_gather(ulines_v, [urow, su + f])
                it = plsc.load_gather(ilines_v, [irow, si + f])
                acc = acc + u * it * ws[f]
            x = acc + bias
            out_v[pl.ds(gg * L, L)] = 1.0 / (1.0 + jnp.exp(-x))
            return carry

        lax.fori_loop(0, HGROUP, group, 0)

    pltpu.sync_copy(out_v, out_hbm.at[pl.ds(base, B_PER_W)])


_gmf = pl.kernel(
    _gmf_body,
    out_type=jax.ShapeDtypeStruct((BATCH,), jnp.float32),
    mesh=plsc.VectorSubcoreMesh(core_axis_name="c", subcore_axis_name="s",
                                num_cores=NC, num_subcores=NS),
    compiler_params=pltpu.CompilerParams(needs_layout_passes=False),
    scratch_types=[
        pltpu.VMEM((NCHUNK, CHUNK), jnp.int32),
        pltpu.VMEM((NCHUNK, CHUNK), jnp.int32),
        pltpu.VMEM((B_PER_W,), jnp.int32),
        pltpu.VMEM((B_PER_W,), jnp.int32),
        pltpu.VMEM((STAGE, 128), jnp.float32),
        pltpu.VMEM((STAGE, 128), jnp.float32),
        pltpu.VMEM((NUM_FACTORS * L,), jnp.float32),
        pltpu.VMEM((L,), jnp.float32),
        pltpu.VMEM((B_PER_W,), jnp.float32),
        pltpu.SemaphoreType.DMA,
    ],
)


def kernel(user_indices, item_indices, user_table, item_table, affine_w, affine_b):
    uidx = user_indices.astype(jnp.int32)
    iidx = item_indices.astype(jnp.int32)
    ut_t = user_table.T
    it_t = item_table.T
    ut_p = _pack(ut_t, ut_t, ut_t, ut_t)
    it_p = _pack(it_t, it_t, it_t, it_t)
    u_tail = user_table[NMAIN:].reshape(NTAIL, 128)
    i_tail = item_table[NMAIN:].reshape(NTAIL, 128)
    w_b = jnp.broadcast_to(affine_w.reshape(NUM_FACTORS, 1),
                           (NUM_FACTORS, L)).reshape(NUM_FACTORS * L)
    b_b = jnp.broadcast_to(affine_b.reshape(1), (L,))
    return _gmf(uidx, iidx, ut_p, it_p, u_tail, i_tail, w_b, b_b)
